# Initial kernel scaffold; baseline (speedup 1.0000x reference)
import jax
import jax.numpy as jnp
from jax.experimental import pallas as pl


def kernel(x, neighbors, weights):
    raise NotImplementedError("write your pallas kernel here")


# speedup vs baseline: 3.0003x; 3.0003x over previous
import functools

import jax
import jax.numpy as jnp
from jax import lax
from jax.experimental import pallas as pl
from jax.experimental.pallas import tpu as pltpu
from jax.experimental.pallas import tpu_sc as plsc

N = 10000
K = 32
D = 128
L = 16
CHUNK_NODES = 4
ROWS = CHUNK_NODES * K
NCHUNKS = N // CHUNK_NODES
NC = 2
NS = 16
NW = NC * NS


def _build():
    mesh = plsc.VectorSubcoreMesh(
        core_axis_name="c", subcore_axis_name="s", num_cores=NC, num_subcores=NS
    )

    @functools.partial(
        pl.kernel,
        mesh=mesh,
        out_type=jax.ShapeDtypeStruct((N, D), jnp.float32),
        scratch_types=[
            pltpu.VMEM((ROWS,), jnp.int32),
            pltpu.VMEM((ROWS,), jnp.float32),
            pltpu.VMEM((ROWS, D), jnp.float32),
            pltpu.VMEM((CHUNK_NODES, D), jnp.float32),
            pltpu.SemaphoreType.DMA,
        ],
        compiler_params=pltpu.CompilerParams(needs_layout_passes=False),
    )
    def body(x_hbm, nbr_hbm, w_hbm, out_hbm, idx_v, w_v, rows_v, out_v, sem):
        wid = lax.axis_index("s") * NC + lax.axis_index("c")
        ntrips = (NCHUNKS - wid + NW - 1) // NW

        def chunk_body(t, carry):
            c = wid + t * NW
            flat = c * ROWS
            pltpu.sync_copy(nbr_hbm.at[pl.ds(flat, ROWS)], idx_v)
            pltpu.sync_copy(w_hbm.at[pl.ds(flat, ROWS)], w_v)
            pltpu.async_copy(x_hbm.at[idx_v], rows_v, sem).wait()
            for n in range(CHUNK_NODES):
                kb = n * K
                w0 = w_v[pl.ds(kb, L)]
                w1 = w_v[pl.ds(kb + L, L)]
                t = w0 + w1
                lane = lax.iota(jnp.int32, L)
                for sh in (8, 4, 2, 1):
                    t = t + t.at[(lane + sh) & (L - 1)].get(
                        mode="promise_in_bounds"
                    )
                inv = jnp.where(t > 0.0, 1.0 / t, 1.0)

                def k_body(k, accs, kb=kb):
                    wk = plsc.load_gather(
                        w_v, [jnp.full((L,), kb + k, jnp.int32)]
                    )
                    r = kb + k
                    return tuple(
                        accs[db] + wk * rows_v[r, pl.ds(db * L, L)]
                        for db in range(D // L)
                    )

                accs = lax.fori_loop(
                    0, K, k_body,
                    tuple(jnp.zeros((L,), jnp.float32) for _ in range(D // L)),
                )
                for db in range(D // L):
                    out_v[n, pl.ds(db * L, L)] = accs[db] * inv
            pltpu.sync_copy(
                out_v, out_hbm.at[pl.ds(c * CHUNK_NODES, CHUNK_NODES)]
            )
            return carry

        lax.fori_loop(0, ntrips, chunk_body, 0)

    return body


_sc_pool = _build()


def kernel(x, neighbors, weights):
    nbr = neighbors.astype(jnp.int32).reshape(-1)
    w = weights.astype(jnp.float32).reshape(-1)
    return _sc_pool(x, nbr, w)


# speedup vs baseline: 4.5855x; 1.5283x over previous
import functools

import jax
import jax.numpy as jnp
from jax import lax
from jax.experimental import pallas as pl
from jax.experimental.pallas import tpu as pltpu
from jax.experimental.pallas import tpu_sc as plsc

N = 10000
K = 32
D = 128
L = 16
DB = D // L
CHUNK_NODES = 4
ROWS = CHUNK_NODES * K
NCHUNKS = N // CHUNK_NODES
NC = 2
NS = 16
NW = NC * NS
NTRIPS = 2 * ((NCHUNKS + NW - 1) // NW + 1) // 2
KU = 8


def _build():
    mesh = plsc.VectorSubcoreMesh(
        core_axis_name="c", subcore_axis_name="s", num_cores=NC, num_subcores=NS
    )

    @functools.partial(
        pl.kernel,
        mesh=mesh,
        out_type=jax.ShapeDtypeStruct((N, D), jnp.float32),
        scratch_types=[
            pltpu.VMEM((ROWS,), jnp.int32),
            pltpu.VMEM((ROWS,), jnp.int32),
            pltpu.VMEM((ROWS,), jnp.float32),
            pltpu.VMEM((ROWS,), jnp.float32),
            pltpu.VMEM((ROWS, D), jnp.float32),
            pltpu.VMEM((ROWS, D), jnp.float32),
            pltpu.VMEM((CHUNK_NODES, D), jnp.float32),
            pltpu.SemaphoreType.DMA,
            pltpu.SemaphoreType.DMA,
        ],
        compiler_params=pltpu.CompilerParams(needs_layout_passes=False),
    )
    def body(x_hbm, nbr_hbm, w_hbm, out_hbm,
             idx_a, idx_b, w_a, w_b, rows_a, rows_b, out_v, sem_a, sem_b):
        wid = lax.axis_index("s") * NC + lax.axis_index("c")

        def clamp(c):
            return jnp.minimum(c, NCHUNKS - 1)

        def fetch(idx_v, w_v, rows_v, sem, c):
            flat = c * ROWS
            pltpu.sync_copy(nbr_hbm.at[pl.ds(flat, ROWS)], idx_v)
            pltpu.sync_copy(w_hbm.at[pl.ds(flat, ROWS)], w_v)
            pltpu.async_copy(x_hbm.at[idx_v], rows_v, sem)

        def wait(idx_v, rows_v, sem):
            pltpu.make_async_copy(x_hbm.at[idx_v], rows_v, sem).wait()

        def reduce_chunk(w_v, rows_v, c):
            for n in range(CHUNK_NODES):
                kb = n * K
                w0 = w_v[pl.ds(kb, L)]
                w1 = w_v[pl.ds(kb + L, L)]
                t = w0 + w1
                lane = lax.iota(jnp.int32, L)
                for sh in (8, 4, 2, 1):
                    t = t + t.at[(lane + sh) & (L - 1)].get(
                        mode="promise_in_bounds"
                    )
                inv = jnp.where(t > 0.0, 1.0 / t, 1.0)

                def k_body(i, accs, kb=kb):
                    for j in range(KU):
                        k = kb + i * KU + j
                        wk = plsc.load_gather(w_v, [jnp.full((L,), k, jnp.int32)])
                        accs = tuple(
                            accs[db] + wk * rows_v[k, pl.ds(db * L, L)]
                            for db in range(DB)
                        )
                    return accs

                accs = lax.fori_loop(
                    0, K // KU, k_body,
                    tuple(jnp.zeros((L,), jnp.float32) for _ in range(DB)),
                )
                for db in range(DB):
                    out_v[n, pl.ds(db * L, L)] = accs[db] * inv
            pltpu.sync_copy(
                out_v, out_hbm.at[pl.ds(c * CHUNK_NODES, CHUNK_NODES)]
            )

        fetch(idx_a, w_a, rows_a, sem_a, wid)

        def pair_body(p, carry):
            c0 = wid + (2 * p) * NW
            c1 = c0 + NW
            fetch(idx_b, w_b, rows_b, sem_b, clamp(c1))
            wait(idx_a, rows_a, sem_a)

            @pl.when(c0 < NCHUNKS)
            def _():
                reduce_chunk(w_a, rows_a, c0)

            fetch(idx_a, w_a, rows_a, sem_a, clamp(c1 + NW))
            wait(idx_b, rows_b, sem_b)

            @pl.when(c1 < NCHUNKS)
            def _():
                reduce_chunk(w_b, rows_b, c1)

            return carry

        lax.fori_loop(0, NTRIPS // 2, pair_body, 0)
        wait(idx_a, rows_a, sem_a)

    return body


_sc_pool = _build()


def kernel(x, neighbors, weights):
    nbr = neighbors.astype(jnp.int32).reshape(-1)
    w = weights.astype(jnp.float32).reshape(-1)
    return _sc_pool(x, nbr, w)
